# baseline (speedup 1.0000x reference)
import jax
import jax.numpy as jnp
from jax.experimental import pallas as pl
from jax.experimental.pallas import tpu as pltpu

B, C_IN, C_OUT, H, W = 64, 192, 192, 14, 14
NUM_EXPERTS, TOP_K = 8, 2
EPS = 1e-5
HW = H * W
RB = 16
MB = 8


def _router_body(x_ref, wr_ref, br_ref, idx_ref, wts_ref):
    pooled = jnp.mean(x_ref[...], axis=2)
    logits = jnp.dot(pooled, wr_ref[...].T,
                     preferred_element_type=jnp.float32) + br_ref[...]
    m = jnp.max(logits, axis=1, keepdims=True)
    e = jnp.exp(logits - m)
    p = e / jnp.sum(e, axis=1, keepdims=True)
    iota = jax.lax.broadcasted_iota(jnp.int32, (RB, NUM_EXPERTS), 1)
    v0 = jnp.max(p, axis=1, keepdims=True)
    i0 = jnp.min(jnp.where(p == v0, iota, NUM_EXPERTS), axis=1, keepdims=True)
    p1 = jnp.where(iota == i0, -jnp.inf, p)
    v1 = jnp.max(p1, axis=1, keepdims=True)
    i1 = jnp.min(jnp.where(p1 == v1, iota, NUM_EXPERTS), axis=1, keepdims=True)
    denom = v0 + v1 + 1e-9
    idx_ref[...] = jnp.concatenate([i0, i1], axis=1).astype(jnp.int32)
    wts_ref[...] = jnp.concatenate([v0 / denom, v1 / denom], axis=1)


def _main_body(idx_ref, wts_ref, x_ref, ws_ref, bs_ref,
               we_ref, be_ref, out_ref):
    base = pl.program_id(0) * MB
    for s in range(MB):
        xb = x_ref[s].astype(jnp.bfloat16)
        i0 = idx_ref[base + s, 0]
        i1 = idx_ref[base + s, 1]
        w0 = wts_ref[base + s, 0]
        w1 = wts_ref[base + s, 1]

        ys = jnp.dot(ws_ref[...], xb, preferred_element_type=jnp.float32)
        ys = ys + bs_ref[...]
        ys = ys * jax.nn.sigmoid(ys)

        def expert(i, wgt):
            y = jnp.dot(we_ref[i], xb, preferred_element_type=jnp.float32)
            y = y + be_ref[i]
            y = y * jax.nn.sigmoid(y)
            return y * wgt

        out_ref[s, :, :HW] = ys + expert(i0, w0) + expert(i1, w1)


def kernel(x, W_r, b_r, W_s, gamma_s, beta_s, W_e, gamma_e, beta_e):
    rs = (1.0 + EPS) ** -0.5
    x2 = x.reshape(B, C_IN, HW)
    idx, wts = pl.pallas_call(
        _router_body,
        grid=(B // RB,),
        in_specs=[
            pl.BlockSpec((RB, C_IN, HW), lambda b: (b, 0, 0)),
            pl.BlockSpec((NUM_EXPERTS, C_IN), lambda b: (0, 0)),
            pl.BlockSpec((1, NUM_EXPERTS), lambda b: (0, 0)),
        ],
        out_specs=[
            pl.BlockSpec((RB, TOP_K), lambda b: (b, 0)),
            pl.BlockSpec((RB, TOP_K), lambda b: (b, 0)),
        ],
        out_shape=[
            jax.ShapeDtypeStruct((B, TOP_K), jnp.int32),
            jax.ShapeDtypeStruct((B, TOP_K), jnp.float32),
        ],
        compiler_params=pltpu.CompilerParams(
            dimension_semantics=("arbitrary",)),
    )(x2, W_r, b_r.reshape(1, NUM_EXPERTS))

    out = pl.pallas_call(
        _main_body,
        grid_spec=pltpu.PrefetchScalarGridSpec(
            num_scalar_prefetch=2,
            grid=(B // MB,),
            in_specs=[
                pl.BlockSpec((MB, C_IN, HW), lambda b, idx_s, wts_s: (b, 0, 0)),
                pl.BlockSpec((C_OUT, C_IN), lambda b, idx_s, wts_s: (0, 0)),
                pl.BlockSpec((C_OUT, 1), lambda b, idx_s, wts_s: (0, 0)),
                pl.BlockSpec((NUM_EXPERTS, C_OUT, C_IN),
                             lambda b, idx_s, wts_s: (0, 0, 0)),
                pl.BlockSpec((NUM_EXPERTS, C_OUT, 1),
                             lambda b, idx_s, wts_s: (0, 0, 0)),
            ],
            out_specs=pl.BlockSpec((MB, C_OUT, 256),
                                   lambda b, idx_s, wts_s: (b, 0, 0)),
        ),
        out_shape=jax.ShapeDtypeStruct((B, C_OUT, 256), jnp.float32),
        compiler_params=pltpu.CompilerParams(
            dimension_semantics=("arbitrary",)),
    )(idx, wts, x2,
      (W_s * (gamma_s * rs)[:, None]).astype(jnp.bfloat16),
      beta_s.reshape(C_OUT, 1),
      (W_e * (gamma_e * rs)[:, :, None]).astype(jnp.bfloat16),
      beta_e.reshape(NUM_EXPERTS, C_OUT, 1))
    return out[:, :, :HW].reshape(B, C_OUT, H, W)


# speedup vs baseline: 1.1753x; 1.1753x over previous
import jax
import jax.numpy as jnp
from jax.experimental import pallas as pl
from jax.experimental.pallas import tpu as pltpu

B, C_IN, C_OUT, H, W = 64, 192, 192, 14, 14
NUM_EXPERTS, TOP_K = 8, 2
EPS = 1e-5
HW = H * W
RB = 16
MB = 8


def _router_body(x_ref, wr_ref, br_ref, idx_ref, wts_ref):
    pooled = jnp.mean(x_ref[...], axis=2)
    logits = jnp.dot(pooled, wr_ref[...].T,
                     preferred_element_type=jnp.float32) + br_ref[...]
    m = jnp.max(logits, axis=1, keepdims=True)
    e = jnp.exp(logits - m)
    p = e / jnp.sum(e, axis=1, keepdims=True)
    iota = jax.lax.broadcasted_iota(jnp.int32, (RB, NUM_EXPERTS), 1)
    v0 = jnp.max(p, axis=1, keepdims=True)
    i0 = jnp.min(jnp.where(p == v0, iota, NUM_EXPERTS), axis=1, keepdims=True)
    p1 = jnp.where(iota == i0, -jnp.inf, p)
    v1 = jnp.max(p1, axis=1, keepdims=True)
    i1 = jnp.min(jnp.where(p1 == v1, iota, NUM_EXPERTS), axis=1, keepdims=True)
    denom = v0 + v1 + 1e-9
    idx_ref[...] = jnp.concatenate([i0, i1], axis=1).astype(jnp.int32)
    wts_ref[...] = jnp.concatenate([v0 / denom, v1 / denom], axis=1)


def _main_body(idx_ref, wts_ref, x_ref, ws_ref, bs_ref,
               we_ref, be_ref, out_ref):
    base = pl.program_id(0) * MB
    for s in range(MB):
        xb = x_ref[s].astype(jnp.bfloat16)
        i0 = idx_ref[base + s, 0]
        i1 = idx_ref[base + s, 1]
        w0 = wts_ref[base + s, 0]
        w1 = wts_ref[base + s, 1]

        ys = jnp.dot(ws_ref[...], xb, preferred_element_type=jnp.float32)
        ys = ys + bs_ref[...]
        ys = ys * jax.nn.sigmoid(ys)

        def expert(i, wgt):
            y = jnp.dot(we_ref[i], xb, preferred_element_type=jnp.float32)
            y = y + be_ref[i]
            y = y * jax.nn.sigmoid(y)
            return y * wgt

        out_ref[s] = (ys + expert(i0, w0) + expert(i1, w1)).astype(jnp.bfloat16)


def kernel(x, W_r, b_r, W_s, gamma_s, beta_s, W_e, gamma_e, beta_e):
    rs = (1.0 + EPS) ** -0.5
    x2 = x.reshape(B, C_IN, HW)
    idx, wts = pl.pallas_call(
        _router_body,
        grid=(B // RB,),
        in_specs=[
            pl.BlockSpec((RB, C_IN, HW), lambda b: (b, 0, 0)),
            pl.BlockSpec((NUM_EXPERTS, C_IN), lambda b: (0, 0)),
            pl.BlockSpec((1, NUM_EXPERTS), lambda b: (0, 0)),
        ],
        out_specs=[
            pl.BlockSpec((RB, TOP_K), lambda b: (b, 0)),
            pl.BlockSpec((RB, TOP_K), lambda b: (b, 0)),
        ],
        out_shape=[
            jax.ShapeDtypeStruct((B, TOP_K), jnp.int32),
            jax.ShapeDtypeStruct((B, TOP_K), jnp.float32),
        ],
        compiler_params=pltpu.CompilerParams(
            dimension_semantics=("arbitrary",)),
    )(x2, W_r, b_r.reshape(1, NUM_EXPERTS))

    out = pl.pallas_call(
        _main_body,
        grid_spec=pltpu.PrefetchScalarGridSpec(
            num_scalar_prefetch=2,
            grid=(B // MB,),
            in_specs=[
                pl.BlockSpec((MB, C_IN, HW), lambda b, idx_s, wts_s: (b, 0, 0)),
                pl.BlockSpec((C_OUT, C_IN), lambda b, idx_s, wts_s: (0, 0)),
                pl.BlockSpec((C_OUT, 1), lambda b, idx_s, wts_s: (0, 0)),
                pl.BlockSpec((NUM_EXPERTS, C_OUT, C_IN),
                             lambda b, idx_s, wts_s: (0, 0, 0)),
                pl.BlockSpec((NUM_EXPERTS, C_OUT, 1),
                             lambda b, idx_s, wts_s: (0, 0, 0)),
            ],
            out_specs=pl.BlockSpec((MB, C_OUT, HW),
                                   lambda b, idx_s, wts_s: (b, 0, 0)),
        ),
        out_shape=jax.ShapeDtypeStruct((B, C_OUT, HW), jnp.bfloat16),
        compiler_params=pltpu.CompilerParams(
            dimension_semantics=("arbitrary",)),
    )(idx, wts, x2,
      (W_s * (gamma_s * rs)[:, None]).astype(jnp.bfloat16),
      beta_s.reshape(C_OUT, 1),
      (W_e * (gamma_e * rs)[:, :, None]).astype(jnp.bfloat16),
      beta_e.reshape(NUM_EXPERTS, C_OUT, 1))
    return out.astype(jnp.float32).reshape(B, C_OUT, H, W)


# speedup vs baseline: 1.1899x; 1.0124x over previous
import jax
import jax.numpy as jnp
from jax.experimental import pallas as pl
from jax.experimental.pallas import tpu as pltpu

B, C_IN, C_OUT, H, W = 64, 192, 192, 14, 14
NUM_EXPERTS, TOP_K = 8, 2
EPS = 1e-5
HW = H * W
RB = 16
MB = 16


def _router_body(x_ref, wr_ref, br_ref, idx_ref, wts_ref):
    pooled = jnp.mean(x_ref[...], axis=2)
    logits = jnp.dot(pooled, wr_ref[...].T,
                     preferred_element_type=jnp.float32) + br_ref[...]
    m = jnp.max(logits, axis=1, keepdims=True)
    e = jnp.exp(logits - m)
    p = e / jnp.sum(e, axis=1, keepdims=True)
    iota = jax.lax.broadcasted_iota(jnp.int32, (RB, NUM_EXPERTS), 1)
    v0 = jnp.max(p, axis=1, keepdims=True)
    i0 = jnp.min(jnp.where(p == v0, iota, NUM_EXPERTS), axis=1, keepdims=True)
    p1 = jnp.where(iota == i0, -jnp.inf, p)
    v1 = jnp.max(p1, axis=1, keepdims=True)
    i1 = jnp.min(jnp.where(p1 == v1, iota, NUM_EXPERTS), axis=1, keepdims=True)
    denom = v0 + v1 + 1e-9
    idx_ref[...] = jnp.concatenate([i0, i1], axis=1).astype(jnp.int32)
    wts_ref[...] = jnp.concatenate([v0 / denom, v1 / denom], axis=1)


def _main_body(idx_ref, wts_ref, x_ref, ws_ref, bs_ref,
               we_ref, be_ref, out_ref):
    base = pl.program_id(0) * MB
    for s in range(MB):
        xb = x_ref[s].astype(jnp.bfloat16)
        i0 = idx_ref[base + s, 0]
        i1 = idx_ref[base + s, 1]
        w0 = wts_ref[base + s, 0]
        w1 = wts_ref[base + s, 1]

        ys = jnp.dot(ws_ref[...], xb, preferred_element_type=jnp.float32)
        ys = ys + bs_ref[...]
        ys = ys * jax.nn.sigmoid(ys)

        def expert(i, wgt):
            y = jnp.dot(we_ref[i], xb, preferred_element_type=jnp.float32)
            y = y + be_ref[i]
            y = y * jax.nn.sigmoid(y)
            return y * wgt

        out_ref[s] = (ys + expert(i0, w0) + expert(i1, w1)).astype(jnp.bfloat16)


def kernel(x, W_r, b_r, W_s, gamma_s, beta_s, W_e, gamma_e, beta_e):
    rs = (1.0 + EPS) ** -0.5
    x2 = x.reshape(B, C_IN, HW)
    idx, wts = pl.pallas_call(
        _router_body,
        grid=(B // RB,),
        in_specs=[
            pl.BlockSpec((RB, C_IN, HW), lambda b: (b, 0, 0)),
            pl.BlockSpec((NUM_EXPERTS, C_IN), lambda b: (0, 0)),
            pl.BlockSpec((1, NUM_EXPERTS), lambda b: (0, 0)),
        ],
        out_specs=[
            pl.BlockSpec((RB, TOP_K), lambda b: (b, 0)),
            pl.BlockSpec((RB, TOP_K), lambda b: (b, 0)),
        ],
        out_shape=[
            jax.ShapeDtypeStruct((B, TOP_K), jnp.int32),
            jax.ShapeDtypeStruct((B, TOP_K), jnp.float32),
        ],
        compiler_params=pltpu.CompilerParams(
            dimension_semantics=("arbitrary",)),
    )(x2, W_r, b_r.reshape(1, NUM_EXPERTS))

    out = pl.pallas_call(
        _main_body,
        grid_spec=pltpu.PrefetchScalarGridSpec(
            num_scalar_prefetch=2,
            grid=(B // MB,),
            in_specs=[
                pl.BlockSpec((MB, C_IN, HW), lambda b, idx_s, wts_s: (b, 0, 0)),
                pl.BlockSpec((C_OUT, C_IN), lambda b, idx_s, wts_s: (0, 0)),
                pl.BlockSpec((C_OUT, 1), lambda b, idx_s, wts_s: (0, 0)),
                pl.BlockSpec((NUM_EXPERTS, C_OUT, C_IN),
                             lambda b, idx_s, wts_s: (0, 0, 0)),
                pl.BlockSpec((NUM_EXPERTS, C_OUT, 1),
                             lambda b, idx_s, wts_s: (0, 0, 0)),
            ],
            out_specs=pl.BlockSpec((MB, C_OUT, HW),
                                   lambda b, idx_s, wts_s: (b, 0, 0)),
        ),
        out_shape=jax.ShapeDtypeStruct((B, C_OUT, HW), jnp.bfloat16),
        compiler_params=pltpu.CompilerParams(
            dimension_semantics=("arbitrary",)),
    )(idx, wts, x2,
      (W_s * (gamma_s * rs)[:, None]).astype(jnp.bfloat16),
      beta_s.reshape(C_OUT, 1),
      (W_e * (gamma_e * rs)[:, :, None]).astype(jnp.bfloat16),
      beta_e.reshape(NUM_EXPERTS, C_OUT, 1))
    return out.astype(jnp.float32).reshape(B, C_OUT, H, W)
